# baseline (speedup 1.0000x reference)
import functools

import jax
import jax.numpy as jnp
from jax import lax
from jax.experimental import pallas as pl
from jax.experimental.pallas import tpu as pltpu
from jax.experimental.pallas import tpu_sc as plsc

ANCHOR_INTERVAL = 16
_B, _H, _S, _D = 4, 16, 4096, 128
_A = _S // ANCHOR_INTERVAL
_G = _B * _H
_R = _G * _A


_mesh = plsc.VectorSubcoreMesh(core_axis_name="c", subcore_axis_name="s")
_NC = 2
_NS = 16
_NW = _NC * _NS
_ROWS_PER_W = _R // _NW
_CHUNK = 128
_NBUF = 4
_NCHUNK = _ROWS_PER_W // _CHUNK


@functools.partial(
    pl.kernel,
    out_type=jax.ShapeDtypeStruct((_R, _D), jnp.float32),
    mesh=_mesh,
    scratch_types=(
        [pltpu.VMEM((_CHUNK, _D), jnp.float32) for _ in range(_NBUF)]
        + [pltpu.SemaphoreType.DMA for _ in range(2 * _NBUF)]
        + [pltpu.VMEM((_NCHUNK, _CHUNK), jnp.int32)]
    ),
)
def _sc_gather(v2, v_out, *scratch):
    bufs = scratch[:_NBUF]
    gsems = scratch[_NBUF:2 * _NBUF]
    ssems = scratch[2 * _NBUF:3 * _NBUF]
    idx = scratch[3 * _NBUF]
    wid = lax.axis_index("s") * _NC + lax.axis_index("c")
    base = wid * _ROWS_PER_W
    n = _NCHUNK

    lanes = ANCHOR_INTERVAL * lax.iota(jnp.int32, 16)
    for c in range(n):
        for g in range(_CHUNK // 16):
            idx[c, pl.ds(g * 16, 16)] = (
                lanes + ANCHOR_INTERVAL * (base + c * _CHUNK + g * 16))

    def start_gather(i):
        return pltpu.async_copy(v2.at[idx.at[i]], bufs[i % _NBUF], gsems[i % _NBUF])

    def start_scatter(i):
        sl = pl.ds(base + i * _CHUNK, _CHUNK)
        return pltpu.async_copy(bufs[i % _NBUF], v_out.at[sl, :], ssems[i % _NBUF])

    gathers = [None] * n
    scatters = [None] * n
    for j in range(min(_NBUF, n)):
        gathers[j] = start_gather(j)
    for i in range(n):
        gathers[i].wait()
        scatters[i] = start_scatter(i)
        if i + _NBUF < n:
            scatters[i].wait()
            gathers[i + _NBUF] = start_gather(i + _NBUF)
    for i in range(max(0, n - _NBUF), n):
        scatters[i].wait()


_Q = 4
_GQ = _G // _Q
_GB = 4


def _tc_body(*refs):
    ins, ko = refs[:_Q], refs[_Q]
    for q in range(_Q):
        ko[q] = ins[q][0, :, :, 0, 0, :]


def _tc_gather(k6):
    in_specs = [
        pl.BlockSpec((1, _GB, _A, 1, 1, _D),
                     functools.partial(lambda q, i: (q, i, 0, 0, 0, 0), q))
        for q in range(_Q)
    ]
    out_spec = pl.BlockSpec((_Q, _GB, _A, _D), lambda i: (0, i, 0, 0))
    return pl.pallas_call(
        _tc_body,
        grid=(_GQ // _GB,),
        in_specs=in_specs,
        out_specs=out_spec,
        out_shape=jax.ShapeDtypeStruct((_Q, _GQ, _A, _D), jnp.float32),
    )(*([k6] * _Q))


def kernel(k, v):
    k6 = k.reshape(_Q, _GQ, _A, ANCHOR_INTERVAL, 1, _D)
    v2 = v.reshape(_R * ANCHOR_INTERVAL, _D)
    ko = _tc_gather(k6)
    vo = _sc_gather(v2)
    return (ko.reshape(_B, _H, _A, _D), vo.reshape(_B, _H, _A, _D))


# speedup vs baseline: 1.0361x; 1.0361x over previous
import functools

import jax
import jax.numpy as jnp
from jax import lax
from jax.experimental import pallas as pl
from jax.experimental.pallas import tpu as pltpu
from jax.experimental.pallas import tpu_sc as plsc

ANCHOR_INTERVAL = 16
_B, _H, _S, _D = 4, 16, 4096, 128
_A = _S // ANCHOR_INTERVAL
_G = _B * _H
_R = _G * _A


_mesh = plsc.VectorSubcoreMesh(core_axis_name="c", subcore_axis_name="s")
_NC = 2
_NS = 16
_NW = _NC * _NS
_ROWS_PER_W = _R // _NW
_CHUNK = 64
_NBUF = 8
_NCHUNK = _ROWS_PER_W // _CHUNK


@functools.partial(
    pl.kernel,
    out_type=jax.ShapeDtypeStruct((_R, _D), jnp.float32),
    mesh=_mesh,
    scratch_types=(
        [pltpu.VMEM((_CHUNK, _D), jnp.float32) for _ in range(_NBUF)]
        + [pltpu.SemaphoreType.DMA for _ in range(2 * _NBUF)]
    ),
)
def _sc_gather(v3, v_out, *scratch):
    bufs = scratch[:_NBUF]
    gsems = scratch[_NBUF:2 * _NBUF]
    ssems = scratch[2 * _NBUF:]
    wid = lax.axis_index("s") * _NC + lax.axis_index("c")
    base = wid * _ROWS_PER_W
    n = _NCHUNK

    def start_gather(i):
        sl = pl.ds(base + i * _CHUNK, _CHUNK)
        return pltpu.async_copy(v3.at[sl, 0, :], bufs[i % _NBUF], gsems[i % _NBUF])

    def start_scatter(i):
        sl = pl.ds(base + i * _CHUNK, _CHUNK)
        return pltpu.async_copy(bufs[i % _NBUF], v_out.at[sl, :], ssems[i % _NBUF])

    gathers = [None] * n
    scatters = [None] * n
    for j in range(min(_NBUF, n)):
        gathers[j] = start_gather(j)
    for i in range(n):
        gathers[i].wait()
        scatters[i] = start_scatter(i)
        if i + _NBUF < n:
            scatters[i].wait()
            gathers[i + _NBUF] = start_gather(i + _NBUF)
    for i in range(max(0, n - _NBUF), n):
        scatters[i].wait()


_Q = 4
_GQ = _G // _Q
_GB = 4


def _tc_body(*refs):
    ins, ko = refs[:_Q], refs[_Q]
    for q in range(_Q):
        ko[q] = ins[q][0, :, :, 0, 0, :]


def _tc_gather(k6):
    in_specs = [
        pl.BlockSpec((1, _GB, _A, 1, 1, _D),
                     functools.partial(lambda q, i: (q, i, 0, 0, 0, 0), q))
        for q in range(_Q)
    ]
    out_spec = pl.BlockSpec((_Q, _GB, _A, _D), lambda i: (0, i, 0, 0))
    return pl.pallas_call(
        _tc_body,
        grid=(_GQ // _GB,),
        in_specs=in_specs,
        out_specs=out_spec,
        out_shape=jax.ShapeDtypeStruct((_Q, _GQ, _A, _D), jnp.float32),
    )(*([k6] * _Q))


def kernel(k, v):
    k6 = k.reshape(_Q, _GQ, _A, ANCHOR_INTERVAL, 1, _D)
    v3 = v.reshape(_R, ANCHOR_INTERVAL, _D)
    ko = _tc_gather(k6)
    vo = _sc_gather(v3)
    return (ko.reshape(_B, _H, _A, _D), vo.reshape(_B, _H, _A, _D))
